# baseline (speedup 1.0000x reference)
import functools

import jax
import jax.numpy as jnp
from jax import lax
from jax.experimental import pallas as pl
from jax.experimental.pallas import tpu as pltpu
from jax.experimental.pallas import tpu_sc as plsc

_NC = 2
_NS = 16
_CHUNK = 128
_RB = 104
_CPW0 = 64
_H = 16
_L = 5


def _sc_mesh():
    return plsc.VectorSubcoreMesh(
        core_axis_name="c", subcore_axis_name="s",
        num_cores=_NC, num_subcores=_NS)


def _sc_degree(src2d, ones128, zeros128, n):
    nchunks = src2d.shape[0]
    cpw = nchunks // (_NC * _NS)
    rps = (n // _NS) // 8 * 8
    rem = n - _NS * rps
    n_acc = n + 16

    def body(src_hbm, ones_hbm, z_hbm, out_hbm, acc, sidx, ones_v,
             sem_a, sem_b):
        c = lax.axis_index("c")
        s = lax.axis_index("s")
        rbase = s * rps
        pltpu.sync_copy(z_hbm.at[pl.ds(rbase, rps)], acc.at[pl.ds(rbase, rps)])

        @pl.when(s == _NS - 1)
        def _():
            pltpu.sync_copy(z_hbm.at[pl.ds(_NS * rps, rem)],
                            acc.at[pl.ds(_NS * rps, rem)])

        pltpu.sync_copy(ones_hbm, ones_v)
        plsc.subcore_barrier()
        wid = c * _NS + s
        pltpu.sync_copy(src_hbm.at[pl.ds(wid * cpw, cpw)], sidx)

        def pair(j, carry):
            da = pltpu.async_copy(ones_v, acc.at[sidx.at[2 * j]], sem_a,
                                  add=True)
            db = pltpu.async_copy(ones_v, acc.at[sidx.at[2 * j + 1]], sem_b,
                                  add=True)
            da.wait()
            db.wait()
            return carry

        lax.fori_loop(0, cpw // 2, pair, 0)
        plsc.subcore_barrier()
        pltpu.sync_copy(acc.at[pl.ds(rbase, rps)],
                        out_hbm.at[pl.ds(c * n + rbase, rps)])

        @pl.when(s == _NS - 1)
        def _():
            pltpu.sync_copy(acc.at[pl.ds(_NS * rps, rem)],
                            out_hbm.at[pl.ds(c * n + _NS * rps, rem)])

    k = pl.kernel(
        body,
        out_type=jax.ShapeDtypeStruct((_NC * n, 16), jnp.float32),
        mesh=_sc_mesh(),
        compiler_params=pltpu.CompilerParams(use_tc_tiling_on_sc=False),
        scratch_types=[
            pltpu.VMEM_SHARED((n_acc, 16), jnp.float32),
            pltpu.VMEM((cpw, _CHUNK), jnp.int32),
            pltpu.VMEM((_CHUNK, 16), jnp.float32),
            pltpu.SemaphoreType.DMA,
            pltpu.SemaphoreType.DMA,
        ],
    )
    return k(src2d, ones128, zeros128)


def _sc_edge(xs, src2d, dst2d, zeros_f, n, cpw0, f, nbuf):
    nchunks = src2d.shape[0]
    cpw_tot = nchunks // _NS
    cpw1 = cpw_tot - cpw0
    cpw_max = max(cpw0, cpw1)
    rps = (n // _NS) // 8 * 8
    rem = n - _NS * rps
    n_acc = n + 16

    def body(xs_hbm, src_hbm, dst_hbm, z_hbm, out_hbm,
             acc, sidx, didx, rows, gsems, ssems):
        c = lax.axis_index("c")
        s = lax.axis_index("s")
        rbase = s * rps
        pltpu.sync_copy(z_hbm.at[pl.ds(rbase, rps)], acc.at[pl.ds(rbase, rps)])

        @pl.when(s == _NS - 1)
        def _():
            pltpu.sync_copy(z_hbm.at[pl.ds(_NS * rps, rem)],
                            acc.at[pl.ds(_NS * rps, rem)])

        plsc.subcore_barrier()
        my_cpw = jnp.where(c == 0, cpw0, cpw1)
        base = c * (_NS * cpw0) + s * my_cpw
        pltpu.sync_copy(src_hbm.at[pl.ds(base, cpw_max)], sidx)
        pltpu.sync_copy(dst_hbm.at[pl.ds(base, cpw_max)], didx)

        def group(j, carry):
            gs = [pltpu.async_copy(xs_hbm.at[sidx.at[nbuf * j + b]], rows[b],
                                   gsems[b]) for b in range(nbuf)]
            ss = []
            for b in range(nbuf):
                gs[b].wait()
                ss.append(pltpu.async_copy(rows[b],
                                           acc.at[didx.at[nbuf * j + b]],
                                           ssems[b], add=True))
            for b in range(nbuf):
                ss[b].wait()
            return carry

        lax.fori_loop(0, my_cpw // nbuf, group, 0)
        plsc.subcore_barrier()
        pltpu.sync_copy(acc.at[pl.ds(rbase, rps)],
                        out_hbm.at[pl.ds(c * n + rbase, rps)])

        @pl.when(s == _NS - 1)
        def _():
            pltpu.sync_copy(acc.at[pl.ds(_NS * rps, rem)],
                            out_hbm.at[pl.ds(c * n + _NS * rps, rem)])

    k = pl.kernel(
        body,
        out_type=jax.ShapeDtypeStruct((_NC * n, f), jnp.float32),
        mesh=_sc_mesh(),
        compiler_params=pltpu.CompilerParams(use_tc_tiling_on_sc=False),
        scratch_types=[
            pltpu.VMEM_SHARED((n_acc, f), jnp.float32),
            pltpu.VMEM((cpw_max, _CHUNK), jnp.int32),
            pltpu.VMEM((cpw_max, _CHUNK), jnp.int32),
            [pltpu.VMEM((_CHUNK, f), jnp.float32) for _ in range(nbuf)],
            [pltpu.SemaphoreType.DMA for _ in range(nbuf)],
            [pltpu.SemaphoreType.DMA for _ in range(nbuf)],
        ],
    )
    return k(xs, src2d, dst2d, zeros_f)


def _dinv_of(deg_ref):
    deg = deg_ref[0, :, 0:1] + deg_ref[1, :, 0:1]
    return lax.rsqrt(jnp.maximum(deg, 1e-12))


def _tc_norm_proj(feat, W_ihT, n, in_f, blk):
    def body(feat_ref, wih_ref, x0_ref, zb_ref):
        x0 = feat_ref[...]
        x0 = x0 / jnp.sum(x0, axis=1, keepdims=True)
        x0_ref[...] = x0
        zb_ref[...] = jnp.dot(x0, wih_ref[...],
                              preferred_element_type=jnp.float32)

    return pl.pallas_call(
        body,
        grid=(n // blk,),
        in_specs=[
            pl.BlockSpec((blk, in_f), lambda i: (i, 0)),
            pl.BlockSpec((in_f, 4 * _H), lambda i: (0, 0)),
        ],
        out_specs=[
            pl.BlockSpec((blk, in_f), lambda i: (i, 0)),
            pl.BlockSpec((blk, 4 * _H), lambda i: (i, 0)),
        ],
        out_shape=[
            jax.ShapeDtypeStruct((n, in_f), jnp.float32),
            jax.ShapeDtypeStruct((n, 4 * _H), jnp.float32),
        ],
    )(feat, W_ihT)


def _tc_lstm(x0, zb, W_hhT, bsum, W0a, W0b, deg2, n, in_f, f0, blk):
    def body(zb_ref, zbp_ref, x0_ref, deg_ref, whh_ref, bs_ref, w0a_ref,
             w0b_ref, out_ref):
        i = pl.program_id(0)
        cur = zb_ref[...]
        prev = zbp_ref[...]
        bs = bs_ref[...]
        h = jnp.zeros((blk, _H), jnp.float32)
        c = jnp.zeros((blk, _H), jnp.float32)
        rowid = i * blk + lax.broadcasted_iota(jnp.int32, (blk, 1), 0)
        for t in range(_L):
            k = _L - t
            shifted = jnp.concatenate([prev[blk - k :], cur[: blk - k]],
                                      axis=0)
            early = jnp.where(rowid > t, cur[t : t + 1], 0.0)
            g = jnp.where(rowid >= _L, shifted, early)
            gates = g + jnp.dot(h, whh_ref[...],
                                preferred_element_type=jnp.float32) + bs
            ig = jax.nn.sigmoid(gates[:, 0 * _H : 1 * _H])
            fg = jax.nn.sigmoid(gates[:, 1 * _H : 2 * _H])
            gg = jnp.tanh(gates[:, 2 * _H : 3 * _H])
            og = jax.nn.sigmoid(gates[:, 3 * _H : 4 * _H])
            c = fg * c + ig * gg
            h = og * jnp.tanh(c)
        xw = (jnp.dot(x0_ref[...], w0a_ref[...],
                      preferred_element_type=jnp.float32)
              + jnp.dot(h, w0b_ref[...], preferred_element_type=jnp.float32))
        out_ref[...] = xw * _dinv_of(deg_ref)

    return pl.pallas_call(
        body,
        grid=(n // blk,),
        in_specs=[
            pl.BlockSpec((blk, 4 * _H), lambda i: (i, 0)),
            pl.BlockSpec((blk, 4 * _H), lambda i: (jnp.maximum(i - 1, 0), 0)),
            pl.BlockSpec((blk, in_f), lambda i: (i, 0)),
            pl.BlockSpec((2, blk, 16), lambda i: (0, i, 0)),
            pl.BlockSpec((_H, 4 * _H), lambda i: (0, 0)),
            pl.BlockSpec((1, 4 * _H), lambda i: (0, 0)),
            pl.BlockSpec((in_f, f0), lambda i: (0, 0)),
            pl.BlockSpec((_H, f0), lambda i: (0, 0)),
        ],
        out_specs=pl.BlockSpec((blk, f0), lambda i: (i, 0)),
        out_shape=jax.ShapeDtypeStruct((n + 8, f0), jnp.float32),
    )(zb, zb, x0, deg2, W_hhT, bsum, W0a, W0b)


def _tc_mid(p, deg2, Wn, bias, n, fin, fout, blk):
    def body(p_ref, deg_ref, w_ref, b_ref, out_ref):
        dinv = _dinv_of(deg_ref)
        y = (p_ref[0] + p_ref[1]) * dinv + b_ref[...]
        hloc = jnp.where(y >= 0, y, 0.01 * y)
        out_ref[...] = jnp.dot(
            hloc, w_ref[...], preferred_element_type=jnp.float32) * dinv

    return pl.pallas_call(
        body,
        grid=(n // blk,),
        in_specs=[
            pl.BlockSpec((2, blk, fin), lambda i: (0, i, 0)),
            pl.BlockSpec((2, blk, 16), lambda i: (0, i, 0)),
            pl.BlockSpec((fin, fout), lambda i: (0, 0)),
            pl.BlockSpec((1, fin), lambda i: (0, 0)),
        ],
        out_specs=pl.BlockSpec((blk, fout), lambda i: (i, 0)),
        out_shape=jax.ShapeDtypeStruct((n + 8, fout), jnp.float32),
    )(p, deg2, Wn, bias)


def _tc_final(p, deg2, bias, n, fout, blk):
    def body(p_ref, deg_ref, b_ref, out_ref):
        dinv = _dinv_of(deg_ref)
        out_ref[...] = (p_ref[0] + p_ref[1]) * dinv + b_ref[...]

    return pl.pallas_call(
        body,
        grid=(n // blk,),
        in_specs=[
            pl.BlockSpec((2, blk, fout), lambda i: (0, i, 0)),
            pl.BlockSpec((2, blk, 16), lambda i: (0, i, 0)),
            pl.BlockSpec((1, fout), lambda i: (0, 0)),
        ],
        out_specs=pl.BlockSpec((blk, fout), lambda i: (i, 0)),
        out_shape=jax.ShapeDtypeStruct((n, fout), jnp.float32),
    )(p, deg2, bias)


def kernel(feat, edge_index, W_ih, W_hh, b_ih, b_hh, W0, b0, W1, b1, W2, b2):
    n, in_f = feat.shape
    e = edge_index.shape[1]
    f0, f1, f2 = W0.shape[1], W1.shape[1], W2.shape[1]
    stride = _NC * _NS * _CHUNK
    ep = ((e + stride - 1) // stride) * stride
    padi = jnp.full((ep - e,), n, jnp.int32)
    src2d = jnp.concatenate([edge_index[0], padi]).reshape(ep // _CHUNK, _CHUNK)
    dst2d = jnp.concatenate([edge_index[1], padi]).reshape(ep // _CHUNK, _CHUNK)

    ones16 = jnp.ones((_CHUNK, 16), jnp.float32)
    z16 = jnp.zeros((n, 16), jnp.float32)
    z0 = jnp.zeros((n, f0), jnp.float32)
    z1 = jnp.zeros((n, f1), jnp.float32)
    z2 = jnp.zeros((n, f2), jnp.float32)

    deg2 = _sc_degree(src2d, ones16, z16, n).reshape(_NC, n, 16)

    W_ihT = W_ih.T
    W_hhT = W_hh.T
    bsum = (b_ih + b_hh).reshape(1, 4 * _H)
    W0a, W0b = W0[:in_f], W0[in_f:]

    x0, zb = _tc_norm_proj(feat, W_ihT, n, in_f, 2000)
    xs0 = _tc_lstm(x0, zb, W_hhT, bsum, W0a, W0b, deg2, n, in_f, f0, 2000)
    p1 = _sc_edge(xs0, src2d, dst2d, z0, n, 56, f0, 2).reshape(_NC, n, f0)
    xs1 = _tc_mid(p1, deg2, W1, b0.reshape(1, f0), n, f0, f1, 2000)
    p2 = _sc_edge(xs1, src2d, dst2d, z1, n, 56, f1, 4).reshape(_NC, n, f1)
    xs2 = _tc_mid(p2, deg2, W2, b1.reshape(1, f1), n, f1, f2, 2000)
    p3 = _sc_edge(xs2, src2d, dst2d, z2, n, 64, f2, 4).reshape(_NC, n, f2)
    return _tc_final(p3, deg2, b2.reshape(1, f2), n, f2, 2000)


# speedup vs baseline: 1.1759x; 1.1759x over previous
import functools

import jax
import jax.numpy as jnp
from jax import lax
from jax.experimental import pallas as pl
from jax.experimental.pallas import tpu as pltpu
from jax.experimental.pallas import tpu_sc as plsc

_NC = 2
_NS = 16
_CHUNK = 128
_RB = 104
_CPW0 = 64
_H = 16
_L = 5


def _sc_mesh():
    return plsc.VectorSubcoreMesh(
        core_axis_name="c", subcore_axis_name="s",
        num_cores=_NC, num_subcores=_NS)


def _sc_degree(src2d, ones128, zeros128, n):
    nchunks = src2d.shape[0]
    cpw = nchunks // (_NC * _NS)
    rps = (n // _NS) // 8 * 8
    rem = n - _NS * rps
    n_acc = n + 16

    def body(src_hbm, ones_hbm, z_hbm, out_hbm, acc, sidx, ones_v,
             sem_a, sem_b):
        c = lax.axis_index("c")
        s = lax.axis_index("s")
        rbase = s * rps
        pltpu.sync_copy(z_hbm.at[pl.ds(rbase, rps)], acc.at[pl.ds(rbase, rps)])

        @pl.when(s == _NS - 1)
        def _():
            pltpu.sync_copy(z_hbm.at[pl.ds(_NS * rps, rem)],
                            acc.at[pl.ds(_NS * rps, rem)])

        pltpu.sync_copy(ones_hbm, ones_v)
        plsc.subcore_barrier()
        wid = c * _NS + s
        pltpu.sync_copy(src_hbm.at[pl.ds(wid * cpw, cpw)], sidx)

        def pair(j, carry):
            da = pltpu.async_copy(ones_v, acc.at[sidx.at[2 * j]], sem_a,
                                  add=True)
            db = pltpu.async_copy(ones_v, acc.at[sidx.at[2 * j + 1]], sem_b,
                                  add=True)
            da.wait()
            db.wait()
            return carry

        lax.fori_loop(0, cpw // 2, pair, 0)
        plsc.subcore_barrier()
        pltpu.sync_copy(acc.at[pl.ds(rbase, rps)],
                        out_hbm.at[pl.ds(c * n + rbase, rps)])

        @pl.when(s == _NS - 1)
        def _():
            pltpu.sync_copy(acc.at[pl.ds(_NS * rps, rem)],
                            out_hbm.at[pl.ds(c * n + _NS * rps, rem)])

    k = pl.kernel(
        body,
        out_type=jax.ShapeDtypeStruct((_NC * n, 16), jnp.float32),
        mesh=_sc_mesh(),
        compiler_params=pltpu.CompilerParams(use_tc_tiling_on_sc=False),
        scratch_types=[
            pltpu.VMEM_SHARED((n_acc, 16), jnp.float32),
            pltpu.VMEM((cpw, _CHUNK), jnp.int32),
            pltpu.VMEM((_CHUNK, 16), jnp.float32),
            pltpu.SemaphoreType.DMA,
            pltpu.SemaphoreType.DMA,
        ],
    )
    return k(src2d, ones128, zeros128)


def _sc_edge(xs, src2d, dst2d, zeros_f, n, cpw0, f, nbuf):
    nchunks = src2d.shape[0]
    cpw_tot = nchunks // _NS
    cpw1 = cpw_tot - cpw0
    cpw_max = max(cpw0, cpw1)
    rps = (n // _NS) // 8 * 8
    rem = n - _NS * rps
    n_acc = n + 16

    def body(xs_hbm, src_hbm, dst_hbm, z_hbm, out_hbm,
             acc, sidx, didx, rows, gsems, ssems):
        c = lax.axis_index("c")
        s = lax.axis_index("s")
        rbase = s * rps
        pltpu.sync_copy(z_hbm.at[pl.ds(rbase, rps)], acc.at[pl.ds(rbase, rps)])

        @pl.when(s == _NS - 1)
        def _():
            pltpu.sync_copy(z_hbm.at[pl.ds(_NS * rps, rem)],
                            acc.at[pl.ds(_NS * rps, rem)])

        plsc.subcore_barrier()
        my_cpw = jnp.where(c == 0, cpw0, cpw1)
        base = c * (_NS * cpw0) + s * my_cpw

        def group(j, carry):
            pltpu.sync_copy(src_hbm.at[pl.ds(base + nbuf * j, nbuf)], sidx)
            pltpu.sync_copy(dst_hbm.at[pl.ds(base + nbuf * j, nbuf)], didx)
            gs = [pltpu.async_copy(xs_hbm.at[sidx.at[b]], rows[b],
                                   gsems[b]) for b in range(nbuf)]
            ss = []
            for b in range(nbuf):
                gs[b].wait()
                ss.append(pltpu.async_copy(rows[b], acc.at[didx.at[b]],
                                           ssems[b], add=True))
            for b in range(nbuf):
                ss[b].wait()
            return carry

        lax.fori_loop(0, my_cpw // nbuf, group, 0)
        plsc.subcore_barrier()
        pltpu.sync_copy(acc.at[pl.ds(rbase, rps)],
                        out_hbm.at[pl.ds(c * n + rbase, rps)])

        @pl.when(s == _NS - 1)
        def _():
            pltpu.sync_copy(acc.at[pl.ds(_NS * rps, rem)],
                            out_hbm.at[pl.ds(c * n + _NS * rps, rem)])

    k = pl.kernel(
        body,
        out_type=jax.ShapeDtypeStruct((_NC * n, f), jnp.float32),
        mesh=_sc_mesh(),
        compiler_params=pltpu.CompilerParams(use_tc_tiling_on_sc=False),
        scratch_types=[
            pltpu.VMEM_SHARED((n_acc, f), jnp.float32),
            pltpu.VMEM((nbuf, _CHUNK), jnp.int32),
            pltpu.VMEM((nbuf, _CHUNK), jnp.int32),
            [pltpu.VMEM((_CHUNK, f), jnp.float32) for _ in range(nbuf)],
            [pltpu.SemaphoreType.DMA for _ in range(nbuf)],
            [pltpu.SemaphoreType.DMA for _ in range(nbuf)],
        ],
    )
    return k(xs, src2d, dst2d, zeros_f)


def _dinv_of(deg_ref):
    deg = deg_ref[0, :, 0:1] + deg_ref[1, :, 0:1]
    return lax.rsqrt(jnp.maximum(deg, 1e-12))


def _tc_norm_proj(feat, W_ihT, n, in_f, blk):
    def body(feat_ref, wih_ref, x0_ref, zb_ref):
        x0 = feat_ref[...]
        x0 = x0 / jnp.sum(x0, axis=1, keepdims=True)
        x0_ref[...] = x0
        zb_ref[...] = jnp.dot(x0, wih_ref[...],
                              preferred_element_type=jnp.float32)

    return pl.pallas_call(
        body,
        grid=(n // blk,),
        in_specs=[
            pl.BlockSpec((blk, in_f), lambda i: (i, 0)),
            pl.BlockSpec((in_f, 4 * _H), lambda i: (0, 0)),
        ],
        out_specs=[
            pl.BlockSpec((blk, in_f), lambda i: (i, 0)),
            pl.BlockSpec((blk, 4 * _H), lambda i: (i, 0)),
        ],
        out_shape=[
            jax.ShapeDtypeStruct((n, in_f), jnp.float32),
            jax.ShapeDtypeStruct((n, 4 * _H), jnp.float32),
        ],
    )(feat, W_ihT)


def _tc_lstm(x0, zb, W_hhT, bsum, W0a, W0b, deg2, n, in_f, f0, blk):
    def body(zb_ref, zbp_ref, x0_ref, deg_ref, whh_ref, bs_ref, w0a_ref,
             w0b_ref, out_ref):
        i = pl.program_id(0)
        cur = zb_ref[...]
        prev = zbp_ref[...]
        bs = bs_ref[...]
        h = jnp.zeros((blk, _H), jnp.float32)
        c = jnp.zeros((blk, _H), jnp.float32)
        rowid = i * blk + lax.broadcasted_iota(jnp.int32, (blk, 1), 0)
        for t in range(_L):
            k = _L - t
            shifted = jnp.concatenate([prev[blk - k :], cur[: blk - k]],
                                      axis=0)
            early = jnp.where(rowid > t, cur[t : t + 1], 0.0)
            g = jnp.where(rowid >= _L, shifted, early)
            gates = g + jnp.dot(h, whh_ref[...],
                                preferred_element_type=jnp.float32) + bs
            ig = jax.nn.sigmoid(gates[:, 0 * _H : 1 * _H])
            fg = jax.nn.sigmoid(gates[:, 1 * _H : 2 * _H])
            gg = jnp.tanh(gates[:, 2 * _H : 3 * _H])
            og = jax.nn.sigmoid(gates[:, 3 * _H : 4 * _H])
            c = fg * c + ig * gg
            h = og * jnp.tanh(c)
        xw = (jnp.dot(x0_ref[...], w0a_ref[...],
                      preferred_element_type=jnp.float32)
              + jnp.dot(h, w0b_ref[...], preferred_element_type=jnp.float32))
        out_ref[...] = xw * _dinv_of(deg_ref)

    return pl.pallas_call(
        body,
        grid=(n // blk,),
        in_specs=[
            pl.BlockSpec((blk, 4 * _H), lambda i: (i, 0)),
            pl.BlockSpec((blk, 4 * _H), lambda i: (jnp.maximum(i - 1, 0), 0)),
            pl.BlockSpec((blk, in_f), lambda i: (i, 0)),
            pl.BlockSpec((2, blk, 16), lambda i: (0, i, 0)),
            pl.BlockSpec((_H, 4 * _H), lambda i: (0, 0)),
            pl.BlockSpec((1, 4 * _H), lambda i: (0, 0)),
            pl.BlockSpec((in_f, f0), lambda i: (0, 0)),
            pl.BlockSpec((_H, f0), lambda i: (0, 0)),
        ],
        out_specs=pl.BlockSpec((blk, f0), lambda i: (i, 0)),
        out_shape=jax.ShapeDtypeStruct((n + 8, f0), jnp.float32),
    )(zb, zb, x0, deg2, W_hhT, bsum, W0a, W0b)


def _tc_mid(p, deg2, Wn, bias, n, fin, fout, blk):
    def body(p_ref, deg_ref, w_ref, b_ref, out_ref):
        dinv = _dinv_of(deg_ref)
        y = (p_ref[0] + p_ref[1]) * dinv + b_ref[...]
        hloc = jnp.where(y >= 0, y, 0.01 * y)
        out_ref[...] = jnp.dot(
            hloc, w_ref[...], preferred_element_type=jnp.float32) * dinv

    return pl.pallas_call(
        body,
        grid=(n // blk,),
        in_specs=[
            pl.BlockSpec((2, blk, fin), lambda i: (0, i, 0)),
            pl.BlockSpec((2, blk, 16), lambda i: (0, i, 0)),
            pl.BlockSpec((fin, fout), lambda i: (0, 0)),
            pl.BlockSpec((1, fin), lambda i: (0, 0)),
        ],
        out_specs=pl.BlockSpec((blk, fout), lambda i: (i, 0)),
        out_shape=jax.ShapeDtypeStruct((n + 8, fout), jnp.float32),
    )(p, deg2, Wn, bias)


def _tc_final(p, deg2, bias, n, fout, blk):
    def body(p_ref, deg_ref, b_ref, out_ref):
        dinv = _dinv_of(deg_ref)
        out_ref[...] = (p_ref[0] + p_ref[1]) * dinv + b_ref[...]

    return pl.pallas_call(
        body,
        grid=(n // blk,),
        in_specs=[
            pl.BlockSpec((2, blk, fout), lambda i: (0, i, 0)),
            pl.BlockSpec((2, blk, 16), lambda i: (0, i, 0)),
            pl.BlockSpec((1, fout), lambda i: (0, 0)),
        ],
        out_specs=pl.BlockSpec((blk, fout), lambda i: (i, 0)),
        out_shape=jax.ShapeDtypeStruct((n, fout), jnp.float32),
    )(p, deg2, bias)


def kernel(feat, edge_index, W_ih, W_hh, b_ih, b_hh, W0, b0, W1, b1, W2, b2):
    n, in_f = feat.shape
    e = edge_index.shape[1]
    f0, f1, f2 = W0.shape[1], W1.shape[1], W2.shape[1]
    stride = _NC * _NS * _CHUNK
    ep = ((e + stride - 1) // stride) * stride
    padi = jnp.full((ep - e,), n, jnp.int32)
    src2d = jnp.concatenate([edge_index[0], padi]).reshape(ep // _CHUNK, _CHUNK)
    dst2d = jnp.concatenate([edge_index[1], padi]).reshape(ep // _CHUNK, _CHUNK)

    ones16 = jnp.ones((_CHUNK, 16), jnp.float32)
    z16 = jnp.zeros((n, 16), jnp.float32)
    z0 = jnp.zeros((n, f0), jnp.float32)
    z1 = jnp.zeros((n, f1), jnp.float32)
    z2 = jnp.zeros((n, f2), jnp.float32)

    deg2 = _sc_degree(src2d, ones16, z16, n).reshape(_NC, n, 16)

    W_ihT = W_ih.T
    W_hhT = W_hh.T
    bsum = (b_ih + b_hh).reshape(1, 4 * _H)
    W0a, W0b = W0[:in_f], W0[in_f:]

    x0, zb = _tc_norm_proj(feat, W_ihT, n, in_f, 2000)
    xs0 = _tc_lstm(x0, zb, W_hhT, bsum, W0a, W0b, deg2, n, in_f, f0, 2000)
    p1 = _sc_edge(xs0, src2d, dst2d, z0, n, 72, f0, 2).reshape(_NC, n, f0)
    xs1 = _tc_mid(p1, deg2, W1, b0.reshape(1, f0), n, f0, f1, 2000)
    p2 = _sc_edge(xs1, src2d, dst2d, z1, n, 72, f1, 4).reshape(_NC, n, f1)
    xs2 = _tc_mid(p2, deg2, W2, b1.reshape(1, f1), n, f1, f2, 2000)
    p3 = _sc_edge(xs2, src2d, dst2d, z2, n, 64, f2, 4).reshape(_NC, n, f2)
    return _tc_final(p3, deg2, b2.reshape(1, f2), n, f2, 2000)
